# Initial kernel scaffold; baseline (speedup 1.0000x reference)
import jax
import jax.numpy as jnp
from jax.experimental import pallas as pl


def kernel(indices, weight):
    raise NotImplementedError("write your pallas kernel here")


# speedup vs baseline: 1.8042x; 1.8042x over previous
import functools

import jax
import jax.numpy as jnp
from jax import lax
from jax.experimental import pallas as pl
from jax.experimental.pallas import tpu as pltpu
from jax.experimental.pallas import tpu_sc as plsc

_NUM_CORES = 2
_NUM_SUBCORES = 16
_LANES = 16
_NUM_WORKERS = _NUM_CORES * _NUM_SUBCORES


@functools.lru_cache(maxsize=None)
def _build(B, L, D, V):
    assert B % _NUM_WORKERS == 0
    assert D % _LANES == 0
    b_per_w = B // _NUM_WORKERS
    mesh = plsc.VectorSubcoreMesh(
        core_axis_name="c", subcore_axis_name="s"
    )

    @functools.partial(
        pl.kernel,
        mesh=mesh,
        out_type=jax.ShapeDtypeStruct((B, D), jnp.float32),
        compiler_params=pltpu.CompilerParams(use_tc_tiling_on_sc=False),
        scratch_types=[
            pltpu.VMEM((b_per_w, L), jnp.int32),
            pltpu.VMEM((L, D), jnp.float32),
            pltpu.VMEM((b_per_w, D), jnp.float32),
            pltpu.SemaphoreType.DMA,
        ],
    )
    def k(idx_hbm, w_hbm, out_hbm, idx_v, rows_v, acc_v, sem):
        wid = lax.axis_index("s") * _NUM_CORES + lax.axis_index("c")
        base = wid * b_per_w
        pltpu.sync_copy(idx_hbm.at[pl.ds(base, b_per_w)], idx_v)

        def body(i, carry):
            pltpu.async_copy(w_hbm.at[idx_v.at[i]], rows_v, sem).wait()
            for d in range(D // _LANES):
                s = pl.ds(d * _LANES, _LANES)
                acc = rows_v[0, s]
                for l in range(1, L):
                    acc = acc + rows_v[l, s]
                acc_v[i, s] = acc
            return carry

        lax.fori_loop(0, b_per_w, body, 0, unroll=False)
        pltpu.sync_copy(acc_v, out_hbm.at[pl.ds(base, b_per_w)])

    return k


def kernel(indices, weight):
    src_shape = indices.shape
    idx2 = indices.reshape(-1, src_shape[-1])
    B, L = idx2.shape
    V, D = weight.shape
    out = _build(B, L, D, V)(idx2, weight)
    return out.reshape(*src_shape[:-1], D)


# speedup vs baseline: 2.0997x; 1.1638x over previous
import functools

import jax
import jax.numpy as jnp
from jax import lax
from jax.experimental import pallas as pl
from jax.experimental.pallas import tpu as pltpu
from jax.experimental.pallas import tpu_sc as plsc

_NUM_CORES = 2
_NUM_SUBCORES = 16
_LANES = 16
_NUM_WORKERS = _NUM_CORES * _NUM_SUBCORES

_C = 2
_NBUF = 4


@functools.lru_cache(maxsize=None)
def _build(B, L, D, V):
    assert B % (_NUM_WORKERS * _C) == 0
    assert D % _LANES == 0
    b_per_w = B // _NUM_WORKERS
    n_chunks = b_per_w // _C
    cl = _C * L
    assert n_chunks % _NBUF == 0
    mesh = plsc.VectorSubcoreMesh(
        core_axis_name="c", subcore_axis_name="s"
    )

    @functools.partial(
        pl.kernel,
        mesh=mesh,
        out_type=jax.ShapeDtypeStruct((B, D), jnp.float32),
        compiler_params=pltpu.CompilerParams(use_tc_tiling_on_sc=False),
        scratch_types=[
            pltpu.VMEM((n_chunks, cl), jnp.int32),
            pltpu.VMEM((_NBUF, cl, D), jnp.float32),
            pltpu.VMEM((b_per_w, D), jnp.float32),
        ]
        + [pltpu.SemaphoreType.DMA] * _NBUF,
    )
    def k(idx_hbm, w_hbm, out_hbm, idx_v, rows_v, acc_v, *sems):
        wid = lax.axis_index("s") * _NUM_CORES + lax.axis_index("c")
        pltpu.sync_copy(idx_hbm.at[pl.ds(wid * n_chunks, n_chunks)], idx_v)

        def gather(c, b):
            return pltpu.make_async_copy(
                w_hbm.at[idx_v.at[c]], rows_v.at[b], sems[b]
            )

        for b in range(_NBUF):
            gather(b, b).start()

        def outer(it, carry):
            g = it * _NBUF
            for b in range(_NBUF):
                c = g + b
                gather(c, b).wait()
                for j in range(_C):
                    for d in range(D // _LANES):
                        s = pl.ds(d * _LANES, _LANES)
                        acc = rows_v[b, j * L, s]
                        for l in range(1, L):
                            acc = acc + rows_v[b, j * L + l, s]
                        acc_v[c * _C + j, s] = acc

                @pl.when(c + _NBUF < n_chunks)
                def _():
                    gather(c + _NBUF, b).start()

            return carry

        lax.fori_loop(0, n_chunks // _NBUF, outer, 0, unroll=False)
        pltpu.sync_copy(acc_v, out_hbm.at[pl.ds(wid * b_per_w, b_per_w)])

    return k


def kernel(indices, weight):
    src_shape = indices.shape
    L = src_shape[-1]
    idx2 = indices.reshape(-1, L)
    B = idx2.shape[0]
    V, D = weight.shape
    idx_chunked = idx2.reshape(B // _C, _C * L)
    out = _build(B, L, D, V)(idx_chunked, weight)
    return out.reshape(*src_shape[:-1], D)
